# baseline (speedup 1.0000x reference)
import functools

import jax
import jax.numpy as jnp
from jax import lax
from jax.experimental import pallas as pl
from jax.experimental.pallas import tpu as pltpu
from jax.experimental.pallas import tpu_sc as plsc

_NC = 2
_NS = 16
_NW = _NC * _NS
_CHUNK = 16
_LANES = 16


def _sc_add_kernel(batch, seq_len, dim, x_hbm, tab_hbm, out_hbm,
                   acc0, acc1, acc2, pos0, pos1,
                   x0sem, x1sem, x2sem, g0sem, g1sem,
                   o0sem, o1sem, o2sem):
    wid = lax.axis_index("s") * _NC + lax.axis_index("c")
    s_per_w = seq_len // _NW
    s_base = wid * s_per_w
    n_chunks = s_per_w // _CHUNK
    total = n_chunks * batch
    csz = _CHUNK * dim

    accs = (acc0, acc1, acc2)
    poss = (pos0, pos1)
    xsems = (x0sem, x1sem, x2sem)
    gsems = (g0sem, g1sem)
    osems = (o0sem, o1sem, o2sem)
    nbuf = len(accs)

    def off_of(it):
        j, b = divmod(it, batch)
        return (b * seq_len + s_base + j * _CHUNK) * dim

    def start_x(it):
        return pltpu.async_copy(
            x_hbm.at[pl.ds(off_of(it), csz)],
            accs[it % nbuf], xsems[it % nbuf])

    def start_tab(j):
        return pltpu.async_copy(
            tab_hbm.at[pl.ds((s_base + j * _CHUNK) * dim, csz)],
            poss[j % 2], gsems[j % 2])

    def start_out(it):
        return pltpu.async_copy(
            accs[it % nbuf], out_hbm.at[pl.ds(off_of(it), csz)],
            osems[it % nbuf])

    def run_add(p, q):
        @plsc.parallel_loop(0, csz, step=_LANES, unroll=8)
        def _(i):
            k = pl.multiple_of(i, _LANES)
            plsc.addupdate(accs[p].at[pl.ds(k, _LANES)],
                           poss[q][pl.ds(k, _LANES)])

    tabs = [start_tab(0)]
    xs = [start_x(0), start_x(1)]
    outs = [None] * nbuf
    for it in range(total):
        p = it % nbuf
        j, b = divmod(it, batch)
        if it + 2 < total:
            p2 = (it + 2) % nbuf
            if outs[p2] is not None:
                outs[p2].wait()
                outs[p2] = None
            xs.append(start_x(it + 2))
        if b == 0 and j + 1 < n_chunks:
            tabs.append(start_tab(j + 1))
        xs[it].wait()
        if b == 0:
            tabs[j].wait()
        run_add(p, j % 2)
        outs[p] = start_out(it)
    for o in outs:
        if o is not None:
            o.wait()


def kernel(x, pos_table):
    batch, seq_len, dim = x.shape
    n = batch * seq_len * dim
    x1d = x.reshape(n)
    tab1d = pos_table.reshape(pos_table.shape[0] * dim)

    mesh = plsc.VectorSubcoreMesh(core_axis_name="c", subcore_axis_name="s")
    run = pl.kernel(
        functools.partial(_sc_add_kernel, batch, seq_len, dim),
        mesh=mesh,
        out_type=jax.ShapeDtypeStruct((n,), jnp.float32),
        scratch_types=[pltpu.VMEM((_CHUNK * dim,), jnp.float32)] * 5
        + [pltpu.SemaphoreType.DMA] * 8,
    )
    out1d = run(x1d, tab1d)
    return out1d.reshape(batch, seq_len, dim)


# speedup vs baseline: 2.7531x; 2.7531x over previous
import functools

import jax
import jax.numpy as jnp
from jax import lax
from jax.experimental import pallas as pl
from jax.experimental.pallas import tpu as pltpu
from jax.experimental.pallas import tpu_sc as plsc

_NC = 2
_NS = 16
_NW = _NC * _NS
_CHUNK = 16
_LANES = 16


def _sc_add_kernel(batch, seq_len, dim, x_hbm, ids_hbm, tab_hbm, out_hbm,
                   idx_all, acc0, acc1, acc2, pos0, pos1,
                   isem, x0sem, x1sem, x2sem, g0sem, g1sem,
                   o0sem, o1sem, o2sem):
    wid = lax.axis_index("s") * _NC + lax.axis_index("c")
    s_per_w = seq_len // _NW
    s_base = wid * s_per_w
    n_chunks = s_per_w // _CHUNK
    total = n_chunks * batch
    vecs = _CHUNK * (dim // _LANES)

    accs = (acc0, acc1, acc2)
    poss = (pos0, pos1)
    xsems = (x0sem, x1sem, x2sem)
    gsems = (g0sem, g1sem)
    osems = (o0sem, o1sem, o2sem)
    nbuf = len(accs)

    pltpu.async_copy(ids_hbm.at[pl.ds(s_base, s_per_w)], idx_all, isem).wait()

    def row_of(it):
        j, b = divmod(it, batch)
        return b * seq_len + s_base + j * _CHUNK

    def start_x(it):
        return pltpu.async_copy(
            x_hbm.at[pl.ds(row_of(it), _CHUNK)],
            accs[it % nbuf], xsems[it % nbuf])

    def start_gather(j):
        return pltpu.async_copy(
            tab_hbm.at[idx_all.at[pl.ds(j * _CHUNK, _CHUNK)]],
            poss[j % 2], gsems[j % 2])

    def start_out(it):
        return pltpu.async_copy(
            accs[it % nbuf], out_hbm.at[pl.ds(row_of(it), _CHUNK)],
            osems[it % nbuf])

    def make_add(p, q):
        def add_body(i, c):
            r = i // (dim // _LANES)
            k = (i % (dim // _LANES)) * _LANES
            plsc.addupdate(accs[p].at[r, pl.ds(k, _LANES)],
                           poss[q][r, pl.ds(k, _LANES)])
            return c
        return add_body

    gathers = [start_gather(0)]
    xs = [start_x(0), start_x(1)]
    outs = [None] * nbuf
    for it in range(total):
        p = it % nbuf
        j, b = divmod(it, batch)
        if it + 2 < total:
            p2 = (it + 2) % nbuf
            if outs[p2] is not None:
                outs[p2].wait()
                outs[p2] = None
            xs.append(start_x(it + 2))
        if b == 0 and j + 1 < n_chunks:
            gathers.append(start_gather(j + 1))
        xs[it].wait()
        if b == 0:
            gathers[j].wait()
        lax.fori_loop(0, vecs, make_add(p, j % 2), 0, unroll=8)
        outs[p] = start_out(it)
    for o in outs:
        if o is not None:
            o.wait()


def kernel(x, pos_table):
    batch, seq_len, dim = x.shape
    rows = batch * seq_len
    x2d = x.reshape(rows, dim)
    position_ids = jnp.arange(seq_len, dtype=jnp.int32)

    mesh = plsc.VectorSubcoreMesh(core_axis_name="c", subcore_axis_name="s")
    run = pl.kernel(
        functools.partial(_sc_add_kernel, batch, seq_len, dim),
        mesh=mesh,
        out_type=jax.ShapeDtypeStruct((rows, dim), jnp.float32),
        scratch_types=[
            pltpu.VMEM((seq_len // _NW,), jnp.int32),
        ] + [pltpu.VMEM((_CHUNK, dim), jnp.float32)] * 5
        + [pltpu.SemaphoreType.DMA] * 9,
    )
    out2d = run(x2d, position_ids, pos_table)
    return out2d.reshape(batch, seq_len, dim)


# speedup vs baseline: 2.7595x; 1.0023x over previous
import functools

import jax
import jax.numpy as jnp
from jax import lax
from jax.experimental import pallas as pl
from jax.experimental.pallas import tpu as pltpu
from jax.experimental.pallas import tpu_sc as plsc

_NC = 2
_NS = 16
_NW = _NC * _NS
_CHUNK = 16
_LANES = 16


def _sc_add_kernel(batch, seq_len, dim, x_hbm, ids_hbm, tab_hbm, out_hbm,
                   idx_all, acc0, acc1, acc2, acc3, pos0, pos1,
                   isem, x0sem, x1sem, x2sem, x3sem, g0sem, g1sem,
                   o0sem, o1sem, o2sem, o3sem):
    wid = lax.axis_index("s") * _NC + lax.axis_index("c")
    s_per_w = seq_len // _NW
    s_base = wid * s_per_w
    n_chunks = s_per_w // _CHUNK
    total = n_chunks * batch
    vecs = _CHUNK * (dim // _LANES)

    accs = (acc0, acc1, acc2, acc3)
    poss = (pos0, pos1)
    xsems = (x0sem, x1sem, x2sem, x3sem)
    gsems = (g0sem, g1sem)
    osems = (o0sem, o1sem, o2sem, o3sem)
    nbuf = len(accs)

    pltpu.async_copy(ids_hbm.at[pl.ds(s_base, s_per_w)], idx_all, isem).wait()

    def row_of(it):
        j, b = divmod(it, batch)
        return b * seq_len + s_base + j * _CHUNK

    def start_x(it):
        return pltpu.async_copy(
            x_hbm.at[pl.ds(row_of(it), _CHUNK)],
            accs[it % nbuf], xsems[it % nbuf])

    def start_gather(j):
        return pltpu.async_copy(
            tab_hbm.at[idx_all.at[pl.ds(j * _CHUNK, _CHUNK)]],
            poss[j % 2], gsems[j % 2])

    def start_out(it):
        return pltpu.async_copy(
            accs[it % nbuf], out_hbm.at[pl.ds(row_of(it), _CHUNK)],
            osems[it % nbuf])

    def make_add(p, q):
        def add_body(i, c):
            r = i // (dim // _LANES)
            k = (i % (dim // _LANES)) * _LANES
            plsc.addupdate(accs[p].at[r, pl.ds(k, _LANES)],
                           poss[q][r, pl.ds(k, _LANES)])
            return c
        return add_body

    gathers = [start_gather(0)]
    xs = [start_x(0), start_x(1), start_x(2)]
    outs = [None] * nbuf
    for it in range(total):
        p = it % nbuf
        j, b = divmod(it, batch)
        if it + 3 < total:
            p2 = (it + 3) % nbuf
            if outs[p2] is not None:
                outs[p2].wait()
                outs[p2] = None
            xs.append(start_x(it + 3))
        if b == 0 and j + 1 < n_chunks:
            gathers.append(start_gather(j + 1))
        xs[it].wait()
        if b == 0:
            gathers[j].wait()
        lax.fori_loop(0, vecs, make_add(p, j % 2), 0, unroll=8)
        outs[p] = start_out(it)
    for o in outs:
        if o is not None:
            o.wait()


def kernel(x, pos_table):
    batch, seq_len, dim = x.shape
    rows = batch * seq_len
    x2d = x.reshape(rows, dim)
    position_ids = jnp.arange(seq_len, dtype=jnp.int32)

    mesh = plsc.VectorSubcoreMesh(core_axis_name="c", subcore_axis_name="s")
    run = pl.kernel(
        functools.partial(_sc_add_kernel, batch, seq_len, dim),
        mesh=mesh,
        out_type=jax.ShapeDtypeStruct((rows, dim), jnp.float32),
        scratch_types=[
            pltpu.VMEM((seq_len // _NW,), jnp.int32),
        ] + [pltpu.VMEM((_CHUNK, dim), jnp.float32)] * 6
        + [pltpu.SemaphoreType.DMA] * 11,
    )
    out2d = run(x2d, position_ids, pos_table)
    return out2d.reshape(batch, seq_len, dim)
